# Initial kernel scaffold; baseline (speedup 1.0000x reference)
import functools

import jax
import jax.numpy as jnp
from jax import lax
from jax.experimental import pallas as pl
from jax.experimental.pallas import tpu as pltpu
from jax.experimental.pallas import tpu_sc as plsc

N = 10000
E = 320000
B = 64

_NT = 16
_EPT = E // _NT
_K = 80
_NCHUNK = _EPT // _K
_ROWS_MAIN = 624
_ROWS_TAIL = N - 16 * _ROWS_MAIN


def _msg_pass_body(h_ref, src_ref, dst_ref, out_ref, sidx, didx, rows, zbuf,
                   acc, sem, *, s):
    row0 = s * _ROWS_MAIN

    def zero_blk(i, _):
        pltpu.sync_copy(zbuf, acc.at[pl.ds(row0 + 8 * i, 8)])
        return 0
    lax.fori_loop(0, _ROWS_MAIN // 8, zero_blk, 0)

    @pl.when(s == _NT - 1)
    def _():
        def zero_tail(i, _):
            pltpu.sync_copy(zbuf, acc.at[pl.ds(16 * _ROWS_MAIN + 8 * i, 8)])
            return 0
        lax.fori_loop(0, _ROWS_TAIL // 8, zero_tail, 0)

    plsc.subcore_barrier()

    base0 = s * _EPT

    def chunk(j, _):
        b = base0 + j * _K
        pltpu.sync_copy(src_ref.at[pl.ds(b, _K)], sidx)
        pltpu.sync_copy(dst_ref.at[pl.ds(b, _K)], didx)
        pltpu.async_copy(h_ref.at[sidx], rows, sem).wait()
        pltpu.sync_copy(rows, acc.at[didx], add=True)
        return 0
    lax.fori_loop(0, _NCHUNK, chunk, 0)

    plsc.subcore_barrier()

    pltpu.sync_copy(acc.at[pl.ds(row0, _ROWS_MAIN)],
                    out_ref.at[pl.ds(row0, _ROWS_MAIN)])

    @pl.when(s == _NT - 1)
    def _():
        pltpu.sync_copy(acc.at[pl.ds(16 * _ROWS_MAIN, _ROWS_TAIL)],
                        out_ref.at[pl.ds(16 * _ROWS_MAIN, _ROWS_TAIL)])


def _make_msg_pass(width):
    mesh = plsc.VectorSubcoreMesh(core_axis_name="c", subcore_axis_name="s")

    @functools.partial(
        pl.kernel,
        out_type=(jax.ShapeDtypeStruct((N, width), jnp.float32),
                  jax.ShapeDtypeStruct((N, width), jnp.float32)),
        mesh=mesh,
        scratch_types=[
            pltpu.VMEM((_K,), jnp.int32),
            pltpu.VMEM((_K,), jnp.int32),
            pltpu.VMEM((_K, width), jnp.float32),
            pltpu.VMEM((8, width), jnp.float32),
            pltpu.VMEM_SHARED((N, width), jnp.float32),
            pltpu.SemaphoreType.DMA,
        ],
    )
    def msg_pass(h0, h1, src, dst, o0, o1, sidx, didx, rows, zbuf, acc, sem):
        c = lax.axis_index("c")
        s = lax.axis_index("s")

        zero16 = jnp.zeros((16,), jnp.float32)

        def zb(i, _):
            r = i // (width // 16)
            col = (i % (width // 16)) * 16
            zbuf[r, pl.ds(col, 16)] = zero16
            return 0
        lax.fori_loop(0, 8 * width // 16, zb, 0)

        @pl.when(c == 0)
        def _():
            _msg_pass_body(h0, src, dst, o0, sidx, didx, rows, zbuf, acc,
                           sem, s=s)

        @pl.when(c == 1)
        def _():
            _msg_pass_body(h1, src, dst, o1, sidx, didx, rows, zbuf, acc,
                           sem, s=s)

    return msg_pass


_msg_pass_128 = _make_msg_pass(128)
_msg_pass_64 = _make_msg_pass(64)


def _lin2_kernel(x_ref, w0_ref, w1_ref, o0_ref, o1_ref, *, relu_in):
    x = x_ref[...]
    if relu_in:
        x = jnp.maximum(x, 0.0)
    o0_ref[...] = jnp.dot(x, w0_ref[...], preferred_element_type=jnp.float32)
    o1_ref[...] = jnp.dot(x, w1_ref[...], preferred_element_type=jnp.float32)


def _lin2_pair_kernel(x0_ref, x1_ref, w0_ref, w1_ref, o0_ref, o1_ref):
    x0 = jnp.maximum(x0_ref[...], 0.0)
    x1 = jnp.maximum(x1_ref[...], 0.0)
    o0_ref[...] = jnp.dot(x0, w0_ref[...], preferred_element_type=jnp.float32)
    o1_ref[...] = jnp.dot(x1, w1_ref[...], preferred_element_type=jnp.float32)


def _lin2(x, w0, w1, relu_in=False):
    d = w0.shape[1]
    return pl.pallas_call(
        functools.partial(_lin2_kernel, relu_in=relu_in),
        out_shape=(jax.ShapeDtypeStruct((N, d), jnp.float32),
                   jax.ShapeDtypeStruct((N, d), jnp.float32)),
    )(x, w0, w1)


def _lin2_pair(x0, x1, w0, w1):
    d = w0.shape[1]
    return pl.pallas_call(
        _lin2_pair_kernel,
        out_shape=(jax.ShapeDtypeStruct((N, d), jnp.float32),
                   jax.ShapeDtypeStruct((N, d), jnp.float32)),
    )(x0, x1, w0, w1)


def _head_kernel(pc0_ref, pc1_ref, ps0_ref, ps1_ref, cb_ref, sb_ref,
                 cw0_ref, cb0_ref, cw1_ref, cb1_ref,
                 sw0_ref, sb0_ref, sw1_ref, sb1_ref,
                 f1w_ref, f1b_ref, f2w_ref, f2b_ref, out_ref):
    iota_b = lax.broadcasted_iota(jnp.int32, (B, N), 0)
    pc = (cb_ref[...] == iota_b).astype(jnp.float32)
    ps = (sb_ref[...] == iota_b).astype(jnp.float32)

    def pool(p, x_ref):
        return jnp.dot(p, jnp.maximum(x_ref[...], 0.0),
                       preferred_element_type=jnp.float32)

    rc0 = pool(pc, pc0_ref)
    rc1 = pool(pc, pc1_ref)
    rs0 = pool(ps, ps0_ref)
    rs1 = pool(ps, ps1_ref)

    def fc(r, w_ref, b_ref):
        return jnp.maximum(
            jnp.dot(r, w_ref[...], preferred_element_type=jnp.float32)
            + b_ref[...], 0.0)

    ind = jnp.concatenate([
        fc(rc0, cw0_ref, cb0_ref),
        fc(rc1, cw1_ref, cb1_ref),
        fc(rs0, sw0_ref, sb0_ref),
        fc(rs1, sw1_ref, sb1_ref),
    ], axis=1)
    hg = jnp.maximum(
        jnp.dot(ind, f1w_ref[...], preferred_element_type=jnp.float32)
        + f1b_ref[...], 0.0)
    out_ref[...] = (jnp.dot(hg, f2w_ref[...],
                            preferred_element_type=jnp.float32)
                    + f2b_ref[...])


def kernel(chr_x, chr_edge_index, chr_x_batch, slv_x, slv_edge_index,
           slv_x_batch, pseudo_batch,
           chr_W00, chr_W01, chr_W10, chr_W11,
           slv_W00, slv_W01, slv_W10, slv_W11,
           cfc_w0, cfc_b0, cfc_w1, cfc_b1,
           sfc_w0, sfc_b0, sfc_w1, sfc_b1,
           fc1_w, fc1_b, fc2_w, fc2_b):
    del pseudo_batch
    pooled = {}
    for pre, x, ei, (Wa0, Wa1, Wb0, Wb1) in (
            ("chr", chr_x, chr_edge_index,
             (chr_W00, chr_W01, chr_W10, chr_W11)),
            ("slv", slv_x, slv_edge_index,
             (slv_W00, slv_W01, slv_W10, slv_W11))):
        src = ei[0]
        dst = ei[1]
        h0, h1 = _lin2(x, Wa0, Wb0)
        m0, m1 = _msg_pass_128(h0, h1, src, dst)
        g0, g1 = _lin2_pair(m0, m1, Wa1, Wb1)
        p0, p1 = _msg_pass_64(g0, g1, src, dst)
        pooled[pre] = (p0, p1)

    out = pl.pallas_call(
        _head_kernel,
        out_shape=jax.ShapeDtypeStruct((B, 1), jnp.float32),
    )(pooled["chr"][0], pooled["chr"][1],
      pooled["slv"][0], pooled["slv"][1],
      chr_x_batch.reshape(1, N), slv_x_batch.reshape(1, N),
      cfc_w0, cfc_b0.reshape(1, -1), cfc_w1, cfc_b1.reshape(1, -1),
      sfc_w0, sfc_b0.reshape(1, -1), sfc_w1, sfc_b1.reshape(1, -1),
      fc1_w, fc1_b.reshape(1, -1), fc2_w, fc2_b.reshape(1, 1))
    return out


# speedup vs baseline: 5.3378x; 5.3378x over previous
import functools

import jax
import jax.numpy as jnp
from jax import lax
from jax.experimental import pallas as pl
from jax.experimental.pallas import tpu as pltpu
from jax.experimental.pallas import tpu_sc as plsc

N = 10000
E = 320000
B = 64
W = 128

_NT = 16
_EPT = E // (2 * _NT)
_K = 80
_NCHUNK = _EPT // _K
_ROWS_MAIN = 624
_ROWS_TAIL = N - 16 * _ROWS_MAIN


def _sc_msg_pass_fn():
    mesh = plsc.VectorSubcoreMesh(core_axis_name="c", subcore_axis_name="s")

    @functools.partial(
        pl.kernel,
        out_type=(jax.ShapeDtypeStruct((N, W), jnp.float32),
                  jax.ShapeDtypeStruct((N, W), jnp.float32)),
        mesh=mesh,
        scratch_types=[
            pltpu.VMEM((_K,), jnp.int32),
            pltpu.VMEM((_K,), jnp.int32),
            pltpu.VMEM((_K, W), jnp.float32),
            pltpu.VMEM((8, W), jnp.float32),
            pltpu.VMEM_SHARED((N, W), jnp.float32),
            pltpu.SemaphoreType.DMA,
        ],
    )
    def msg_pass(h, src, dst, o_a, o_b, sidx, didx, rows, zbuf, acc, sem):
        c = lax.axis_index("c")
        s = lax.axis_index("s")
        row0 = s * _ROWS_MAIN

        zero16 = jnp.zeros((16,), jnp.float32)

        def zb(i, _):
            r = i // (W // 16)
            col = (i % (W // 16)) * 16
            zbuf[r, pl.ds(col, 16)] = zero16
            return 0
        lax.fori_loop(0, 8 * W // 16, zb, 0)

        def zero_blk(i, _):
            pltpu.sync_copy(zbuf, acc.at[pl.ds(row0 + 8 * i, 8)])
            return 0
        lax.fori_loop(0, _ROWS_MAIN // 8, zero_blk, 0)

        @pl.when(s == _NT - 1)
        def _():
            def zero_tail(i, _):
                pltpu.sync_copy(zbuf, acc.at[pl.ds(16 * _ROWS_MAIN + 8 * i, 8)])
                return 0
            lax.fori_loop(0, _ROWS_TAIL // 8, zero_tail, 0)

        plsc.subcore_barrier()

        base0 = (c * _NT + s) * _EPT

        def chunk(j, _):
            b = base0 + j * _K
            pltpu.sync_copy(src.at[pl.ds(b, _K)], sidx)
            pltpu.sync_copy(dst.at[pl.ds(b, _K)], didx)
            pltpu.async_copy(h.at[sidx], rows, sem).wait()
            pltpu.sync_copy(rows, acc.at[didx], add=True)
            return 0
        lax.fori_loop(0, _NCHUNK, chunk, 0)

        plsc.subcore_barrier()

        @pl.when(c == 0)
        def _():
            pltpu.sync_copy(acc.at[pl.ds(row0, _ROWS_MAIN)],
                            o_a.at[pl.ds(row0, _ROWS_MAIN)])

            @pl.when(s == _NT - 1)
            def _():
                pltpu.sync_copy(acc.at[pl.ds(16 * _ROWS_MAIN, _ROWS_TAIL)],
                                o_a.at[pl.ds(16 * _ROWS_MAIN, _ROWS_TAIL)])

        @pl.when(c == 1)
        def _():
            pltpu.sync_copy(acc.at[pl.ds(row0, _ROWS_MAIN)],
                            o_b.at[pl.ds(row0, _ROWS_MAIN)])

            @pl.when(s == _NT - 1)
            def _():
                pltpu.sync_copy(acc.at[pl.ds(16 * _ROWS_MAIN, _ROWS_TAIL)],
                                o_b.at[pl.ds(16 * _ROWS_MAIN, _ROWS_TAIL)])

    return msg_pass


_SC_CACHE = {}


def _msg_pass():
    if "mp" not in _SC_CACHE:
        _SC_CACHE["mp"] = _sc_msg_pass_fn()
    return _SC_CACHE["mp"]


def _mid_kernel(za_ref, zb_ref, w00_ref, w10_ref, w01_ref, w11_ref, g_ref):
    z = za_ref[...] + zb_ref[...]
    c0 = jnp.maximum(jnp.dot(z, w00_ref[...],
                             preferred_element_type=jnp.float32), 0.0)
    c1 = jnp.maximum(jnp.dot(z, w10_ref[...],
                             preferred_element_type=jnp.float32), 0.0)
    g_ref[...] = jnp.concatenate(
        [jnp.dot(c0, w01_ref[...], preferred_element_type=jnp.float32),
         jnp.dot(c1, w11_ref[...], preferred_element_type=jnp.float32)],
        axis=1)


def _mid(za, zb, w00, w10, w01, w11):
    return pl.pallas_call(
        _mid_kernel,
        out_shape=jax.ShapeDtypeStruct((N, W), jnp.float32),
    )(za, zb, w00, w10, w01, w11)


def _head_kernel(pca_ref, pcb_ref, psa_ref, psb_ref, cb_ref, sb_ref,
                 cw0_ref, cb0_ref, cw1_ref, cb1_ref,
                 sw0_ref, sb0_ref, sw1_ref, sb1_ref,
                 f1w_ref, f1b_ref, f2w_ref, f2b_ref, out_ref):
    iota_b = lax.broadcasted_iota(jnp.int32, (B, N), 0)
    pc = (cb_ref[...] == iota_b).astype(jnp.float32)
    ps = (sb_ref[...] == iota_b).astype(jnp.float32)

    m2c = jnp.maximum(pca_ref[...] + pcb_ref[...], 0.0)
    m2s = jnp.maximum(psa_ref[...] + psb_ref[...], 0.0)

    repc = jnp.dot(pc, m2c, preferred_element_type=jnp.float32)
    reps = jnp.dot(ps, m2s, preferred_element_type=jnp.float32)

    def fc(r, w_ref, b_ref):
        return jnp.maximum(
            jnp.dot(r, w_ref[...], preferred_element_type=jnp.float32)
            + b_ref[...], 0.0)

    ind = jnp.concatenate([
        fc(repc[:, :64], cw0_ref, cb0_ref),
        fc(repc[:, 64:], cw1_ref, cb1_ref),
        fc(reps[:, :64], sw0_ref, sb0_ref),
        fc(reps[:, 64:], sw1_ref, sb1_ref),
    ], axis=1)
    hg = jnp.maximum(
        jnp.dot(ind, f1w_ref[...], preferred_element_type=jnp.float32)
        + f1b_ref[...], 0.0)
    out_ref[...] = (jnp.dot(hg, f2w_ref[...],
                            preferred_element_type=jnp.float32)
                    + f2b_ref[...])


def kernel(chr_x, chr_edge_index, chr_x_batch, slv_x, slv_edge_index,
           slv_x_batch, pseudo_batch,
           chr_W00, chr_W01, chr_W10, chr_W11,
           slv_W00, slv_W01, slv_W10, slv_W11,
           cfc_w0, cfc_b0, cfc_w1, cfc_b1,
           sfc_w0, sfc_b0, sfc_w1, sfc_b1,
           fc1_w, fc1_b, fc2_w, fc2_b):
    del pseudo_batch
    mp = _msg_pass()
    pooled = {}
    for pre, x, ei, (W00, W01, W10, W11) in (
            ("chr", chr_x, chr_edge_index,
             (chr_W00, chr_W01, chr_W10, chr_W11)),
            ("slv", slv_x, slv_edge_index,
             (slv_W00, slv_W01, slv_W10, slv_W11))):
        src = ei[0]
        dst = ei[1]
        za, zb = mp(x, src, dst)
        g = _mid(za, zb, W00, W10, W01, W11)
        pa, pb = mp(g, src, dst)
        pooled[pre] = (pa, pb)

    out = pl.pallas_call(
        _head_kernel,
        out_shape=jax.ShapeDtypeStruct((B, 1), jnp.float32),
    )(pooled["chr"][0], pooled["chr"][1],
      pooled["slv"][0], pooled["slv"][1],
      chr_x_batch.reshape(1, N), slv_x_batch.reshape(1, N),
      cfc_w0, cfc_b0.reshape(1, -1), cfc_w1, cfc_b1.reshape(1, -1),
      sfc_w0, sfc_b0.reshape(1, -1), sfc_w1, sfc_b1.reshape(1, -1),
      fc1_w, fc1_b.reshape(1, -1), fc2_w, fc2_b.reshape(1, 1))
    return out


# speedup vs baseline: 8.7548x; 1.6401x over previous
import functools

import jax
import jax.numpy as jnp
from jax import lax
from jax.experimental import pallas as pl
from jax.experimental.pallas import tpu as pltpu
from jax.experimental.pallas import tpu_sc as plsc

N = 10000
E = 320000
B = 64
W = 128

_NT = 16
_EPT = E // (2 * _NT)
_K = 80
_NCHUNK = _EPT // _K
_ROWS_MAIN = 624
_ROWS_TAIL = N - 16 * _ROWS_MAIN


def _sc_msg_pass_fn():
    mesh = plsc.VectorSubcoreMesh(core_axis_name="c", subcore_axis_name="s")

    @functools.partial(
        pl.kernel,
        out_type=(jax.ShapeDtypeStruct((N, W), jnp.float32),
                  jax.ShapeDtypeStruct((N, W), jnp.float32)),
        mesh=mesh,
        scratch_types=[
            pltpu.VMEM((_K,), jnp.int32),
            pltpu.VMEM((_K,), jnp.int32),
            pltpu.VMEM((_K,), jnp.int32),
            pltpu.VMEM((_K,), jnp.int32),
            pltpu.VMEM((_K, W), jnp.float32),
            pltpu.VMEM((_K, W), jnp.float32),
            pltpu.VMEM((8, W), jnp.float32),
            pltpu.VMEM_SHARED((N, W), jnp.float32),
            pltpu.SemaphoreType.DMA,
            pltpu.SemaphoreType.DMA,
        ],
    )
    def msg_pass(h, src, dst, o_a, o_b, si0, di0, si1, di1, rows0, rows1,
                 zbuf, acc, sem0, sem1):
        c = lax.axis_index("c")
        s = lax.axis_index("s")
        row0 = s * _ROWS_MAIN

        zero16 = jnp.zeros((16,), jnp.float32)

        def zb(i, _):
            r = i // (W // 16)
            col = (i % (W // 16)) * 16
            zbuf[r, pl.ds(col, 16)] = zero16
            return 0
        lax.fori_loop(0, 8 * W // 16, zb, 0)

        def zero_blk(i, _):
            pltpu.sync_copy(zbuf, acc.at[pl.ds(row0 + 8 * i, 8)])
            return 0
        lax.fori_loop(0, _ROWS_MAIN // 8, zero_blk, 0)

        @pl.when(s == _NT - 1)
        def _():
            def zero_tail(i, _):
                pltpu.sync_copy(zbuf, acc.at[pl.ds(16 * _ROWS_MAIN + 8 * i, 8)])
                return 0
            lax.fori_loop(0, _ROWS_TAIL // 8, zero_tail, 0)

        plsc.subcore_barrier()

        base0 = (c * _NT + s) * _EPT

        def issue(j, si, di, rows, sem):
            b = base0 + j * _K
            pltpu.sync_copy(src.at[pl.ds(b, _K)], si)
            pltpu.sync_copy(dst.at[pl.ds(b, _K)], di)
            pltpu.async_copy(h.at[si], rows, sem)

        def step(j, si, di, rows, sem, si_n, di_n, rows_n, sem_n):
            @pl.when(j + 1 < _NCHUNK)
            def _():
                issue(j + 1, si_n, di_n, rows_n, sem_n)
            pltpu.make_async_copy(h.at[si], rows, sem).wait()
            pltpu.sync_copy(rows, acc.at[di], add=True)

        issue(0, si0, di0, rows0, sem0)

        def chunk(j, _):
            @pl.when(j % 2 == 0)
            def _():
                step(j, si0, di0, rows0, sem0, si1, di1, rows1, sem1)

            @pl.when(j % 2 == 1)
            def _():
                step(j, si1, di1, rows1, sem1, si0, di0, rows0, sem0)
            return 0
        lax.fori_loop(0, _NCHUNK, chunk, 0)

        plsc.subcore_barrier()

        @pl.when(c == 0)
        def _():
            pltpu.sync_copy(acc.at[pl.ds(row0, _ROWS_MAIN)],
                            o_a.at[pl.ds(row0, _ROWS_MAIN)])

            @pl.when(s == _NT - 1)
            def _():
                pltpu.sync_copy(acc.at[pl.ds(16 * _ROWS_MAIN, _ROWS_TAIL)],
                                o_a.at[pl.ds(16 * _ROWS_MAIN, _ROWS_TAIL)])

        @pl.when(c == 1)
        def _():
            pltpu.sync_copy(acc.at[pl.ds(row0, _ROWS_MAIN)],
                            o_b.at[pl.ds(row0, _ROWS_MAIN)])

            @pl.when(s == _NT - 1)
            def _():
                pltpu.sync_copy(acc.at[pl.ds(16 * _ROWS_MAIN, _ROWS_TAIL)],
                                o_b.at[pl.ds(16 * _ROWS_MAIN, _ROWS_TAIL)])

    return msg_pass


_SC_CACHE = {}


def _msg_pass():
    if "mp" not in _SC_CACHE:
        _SC_CACHE["mp"] = _sc_msg_pass_fn()
    return _SC_CACHE["mp"]


def _mid_kernel(za_ref, zb_ref, w00_ref, w10_ref, w01_ref, w11_ref, g_ref):
    z = za_ref[...] + zb_ref[...]
    c0 = jnp.maximum(jnp.dot(z, w00_ref[...],
                             preferred_element_type=jnp.float32), 0.0)
    c1 = jnp.maximum(jnp.dot(z, w10_ref[...],
                             preferred_element_type=jnp.float32), 0.0)
    g_ref[...] = jnp.concatenate(
        [jnp.dot(c0, w01_ref[...], preferred_element_type=jnp.float32),
         jnp.dot(c1, w11_ref[...], preferred_element_type=jnp.float32)],
        axis=1)


def _mid(za, zb, w00, w10, w01, w11):
    return pl.pallas_call(
        _mid_kernel,
        out_shape=jax.ShapeDtypeStruct((N, W), jnp.float32),
    )(za, zb, w00, w10, w01, w11)


def _head_kernel(pca_ref, pcb_ref, psa_ref, psb_ref, cb_ref, sb_ref,
                 cw0_ref, cb0_ref, cw1_ref, cb1_ref,
                 sw0_ref, sb0_ref, sw1_ref, sb1_ref,
                 f1w_ref, f1b_ref, f2w_ref, f2b_ref, out_ref):
    iota_b = lax.broadcasted_iota(jnp.int32, (B, N), 0)
    pc = (cb_ref[...] == iota_b).astype(jnp.float32)
    ps = (sb_ref[...] == iota_b).astype(jnp.float32)

    m2c = jnp.maximum(pca_ref[...] + pcb_ref[...], 0.0)
    m2s = jnp.maximum(psa_ref[...] + psb_ref[...], 0.0)

    repc = jnp.dot(pc, m2c, preferred_element_type=jnp.float32)
    reps = jnp.dot(ps, m2s, preferred_element_type=jnp.float32)

    def fc(r, w_ref, b_ref):
        return jnp.maximum(
            jnp.dot(r, w_ref[...], preferred_element_type=jnp.float32)
            + b_ref[...], 0.0)

    ind = jnp.concatenate([
        fc(repc[:, :64], cw0_ref, cb0_ref),
        fc(repc[:, 64:], cw1_ref, cb1_ref),
        fc(reps[:, :64], sw0_ref, sb0_ref),
        fc(reps[:, 64:], sw1_ref, sb1_ref),
    ], axis=1)
    hg = jnp.maximum(
        jnp.dot(ind, f1w_ref[...], preferred_element_type=jnp.float32)
        + f1b_ref[...], 0.0)
    out_ref[...] = (jnp.dot(hg, f2w_ref[...],
                            preferred_element_type=jnp.float32)
                    + f2b_ref[...])


def kernel(chr_x, chr_edge_index, chr_x_batch, slv_x, slv_edge_index,
           slv_x_batch, pseudo_batch,
           chr_W00, chr_W01, chr_W10, chr_W11,
           slv_W00, slv_W01, slv_W10, slv_W11,
           cfc_w0, cfc_b0, cfc_w1, cfc_b1,
           sfc_w0, sfc_b0, sfc_w1, sfc_b1,
           fc1_w, fc1_b, fc2_w, fc2_b):
    del pseudo_batch
    mp = _msg_pass()
    pooled = {}
    for pre, x, ei, (W00, W01, W10, W11) in (
            ("chr", chr_x, chr_edge_index,
             (chr_W00, chr_W01, chr_W10, chr_W11)),
            ("slv", slv_x, slv_edge_index,
             (slv_W00, slv_W01, slv_W10, slv_W11))):
        src = ei[0]
        dst = ei[1]
        za, zb = mp(x, src, dst)
        g = _mid(za, zb, W00, W10, W01, W11)
        pa, pb = mp(g, src, dst)
        pooled[pre] = (pa, pb)

    out = pl.pallas_call(
        _head_kernel,
        out_shape=jax.ShapeDtypeStruct((B, 1), jnp.float32),
    )(pooled["chr"][0], pooled["chr"][1],
      pooled["slv"][0], pooled["slv"][1],
      chr_x_batch.reshape(1, N), slv_x_batch.reshape(1, N),
      cfc_w0, cfc_b0.reshape(1, -1), cfc_w1, cfc_b1.reshape(1, -1),
      sfc_w0, sfc_b0.reshape(1, -1), sfc_w1, sfc_b1.reshape(1, -1),
      fc1_w, fc1_b.reshape(1, -1), fc2_w, fc2_b.reshape(1, 1))
    return out


# speedup vs baseline: 13.0797x; 1.4940x over previous
import functools

import jax
import jax.numpy as jnp
from jax import lax
from jax.experimental import pallas as pl
from jax.experimental.pallas import tpu as pltpu
from jax.experimental.pallas import tpu_sc as plsc

N = 10000
E = 320000
B = 64
W = 128

_NT = 16
_EPT = E // (2 * _NT)
_K = 80
_NCHUNK = _EPT // _K
_ROWS_MAIN = 624
_ROWS_TAIL = N - 16 * _ROWS_MAIN


def _sc_msg_pass_fn():
    mesh = plsc.VectorSubcoreMesh(core_axis_name="c", subcore_axis_name="s")

    @functools.partial(
        pl.kernel,
        out_type=(jax.ShapeDtypeStruct((N, W), jnp.float32),
                  jax.ShapeDtypeStruct((N, W), jnp.float32)),
        mesh=mesh,
        scratch_types=[
            pltpu.VMEM((_K,), jnp.int32),
            pltpu.VMEM((_K,), jnp.int32),
            pltpu.VMEM((_K,), jnp.int32),
            pltpu.VMEM((_K,), jnp.int32),
            pltpu.VMEM((_K,), jnp.int32),
            pltpu.VMEM((_K,), jnp.int32),
            pltpu.VMEM((_K, W), jnp.float32),
            pltpu.VMEM((_K, W), jnp.float32),
            pltpu.VMEM((_K, W), jnp.float32),
            pltpu.VMEM((8, W), jnp.float32),
            pltpu.VMEM_SHARED((N, W), jnp.float32),
        ] + [pltpu.SemaphoreType.DMA] * 9,
    )
    def msg_pass(h, src, dst, o_a, o_b,
                 si0, di0, si1, di1, si2, di2, rows0, rows1, rows2,
                 zbuf, acc,
                 ss0, ss1, ss2, ds0, ds1, ds2, gs0, gs1, gs2):
        c = lax.axis_index("c")
        s = lax.axis_index("s")
        row0 = s * _ROWS_MAIN

        zero16 = jnp.zeros((16,), jnp.float32)

        def zb(i, _):
            r = i // (W // 16)
            col = (i % (W // 16)) * 16
            zbuf[r, pl.ds(col, 16)] = zero16
            return 0
        lax.fori_loop(0, 8 * W // 16, zb, 0)

        def zero_blk(i, _):
            pltpu.sync_copy(zbuf, acc.at[pl.ds(row0 + 8 * i, 8)])
            return 0
        lax.fori_loop(0, _ROWS_MAIN // 8, zero_blk, 0)

        @pl.when(s == _NT - 1)
        def _():
            def zero_tail(i, _):
                pltpu.sync_copy(zbuf, acc.at[pl.ds(16 * _ROWS_MAIN + 8 * i, 8)])
                return 0
            lax.fori_loop(0, _ROWS_TAIL // 8, zero_tail, 0)

        plsc.subcore_barrier()

        base0 = (c * _NT + s) * _EPT
        bufs = ((si0, di0, rows0, ss0, ds0, gs0),
                (si1, di1, rows1, ss1, ds1, gs1),
                (si2, di2, rows2, ss2, ds2, gs2))

        def load_idx(j, b):
            si, di, ss, ds, _, _ = b[0], b[1], b[3], b[4], None, None
            off = base0 + j * _K
            pltpu.async_copy(src.at[pl.ds(off, _K)], si, ss)
            pltpu.async_copy(dst.at[pl.ds(off, _K)], di, ds)

        def start_gather(j, b):
            si, di, rows, ss, ds, gs = b
            off = base0 + j * _K
            pltpu.make_async_copy(src.at[pl.ds(off, _K)], si, ss).wait()
            pltpu.make_async_copy(dst.at[pl.ds(off, _K)], di, ds).wait()
            pltpu.async_copy(h.at[si], rows, gs)

        def finish(j, b):
            si, di, rows, ss, ds, gs = b
            pltpu.make_async_copy(h.at[si], rows, gs).wait()
            pltpu.sync_copy(rows, acc.at[di], add=True)

        def step(j, b_cur, b_nxt, b_fut):
            @pl.when(j + 2 < _NCHUNK)
            def _():
                load_idx(j + 2, b_fut)

            @pl.when(j + 1 < _NCHUNK)
            def _():
                start_gather(j + 1, b_nxt)
            finish(j, b_cur)

        load_idx(0, bufs[0])
        load_idx(1, bufs[1])
        start_gather(0, bufs[0])

        def chunk(j, _):
            @pl.when(j % 3 == 0)
            def _():
                step(j, bufs[0], bufs[1], bufs[2])

            @pl.when(j % 3 == 1)
            def _():
                step(j, bufs[1], bufs[2], bufs[0])

            @pl.when(j % 3 == 2)
            def _():
                step(j, bufs[2], bufs[0], bufs[1])
            return 0
        lax.fori_loop(0, _NCHUNK, chunk, 0)

        plsc.subcore_barrier()

        @pl.when(c == 0)
        def _():
            pltpu.sync_copy(acc.at[pl.ds(row0, _ROWS_MAIN)],
                            o_a.at[pl.ds(row0, _ROWS_MAIN)])

            @pl.when(s == _NT - 1)
            def _():
                pltpu.sync_copy(acc.at[pl.ds(16 * _ROWS_MAIN, _ROWS_TAIL)],
                                o_a.at[pl.ds(16 * _ROWS_MAIN, _ROWS_TAIL)])

        @pl.when(c == 1)
        def _():
            pltpu.sync_copy(acc.at[pl.ds(row0, _ROWS_MAIN)],
                            o_b.at[pl.ds(row0, _ROWS_MAIN)])

            @pl.when(s == _NT - 1)
            def _():
                pltpu.sync_copy(acc.at[pl.ds(16 * _ROWS_MAIN, _ROWS_TAIL)],
                                o_b.at[pl.ds(16 * _ROWS_MAIN, _ROWS_TAIL)])

    return msg_pass


_SC_CACHE = {}


def _msg_pass():
    if "mp" not in _SC_CACHE:
        _SC_CACHE["mp"] = _sc_msg_pass_fn()
    return _SC_CACHE["mp"]


def _mid_kernel(za_ref, zb_ref, w00_ref, w10_ref, w01_ref, w11_ref, g_ref):
    z = za_ref[...] + zb_ref[...]
    c0 = jnp.maximum(jnp.dot(z, w00_ref[...],
                             preferred_element_type=jnp.float32), 0.0)
    c1 = jnp.maximum(jnp.dot(z, w10_ref[...],
                             preferred_element_type=jnp.float32), 0.0)
    g_ref[...] = jnp.concatenate(
        [jnp.dot(c0, w01_ref[...], preferred_element_type=jnp.float32),
         jnp.dot(c1, w11_ref[...], preferred_element_type=jnp.float32)],
        axis=1)


def _mid(za, zb, w00, w10, w01, w11):
    return pl.pallas_call(
        _mid_kernel,
        out_shape=jax.ShapeDtypeStruct((N, W), jnp.float32),
    )(za, zb, w00, w10, w01, w11)


def _head_kernel(pca_ref, pcb_ref, psa_ref, psb_ref, cb_ref, sb_ref,
                 cw0_ref, cb0_ref, cw1_ref, cb1_ref,
                 sw0_ref, sb0_ref, sw1_ref, sb1_ref,
                 f1w_ref, f1b_ref, f2w_ref, f2b_ref, out_ref):
    iota_b = lax.broadcasted_iota(jnp.int32, (B, N), 0)
    pc = (cb_ref[...] == iota_b).astype(jnp.float32)
    ps = (sb_ref[...] == iota_b).astype(jnp.float32)

    m2c = jnp.maximum(pca_ref[...] + pcb_ref[...], 0.0)
    m2s = jnp.maximum(psa_ref[...] + psb_ref[...], 0.0)

    repc = jnp.dot(pc, m2c, preferred_element_type=jnp.float32)
    reps = jnp.dot(ps, m2s, preferred_element_type=jnp.float32)

    def fc(r, w_ref, b_ref):
        return jnp.maximum(
            jnp.dot(r, w_ref[...], preferred_element_type=jnp.float32)
            + b_ref[...], 0.0)

    ind = jnp.concatenate([
        fc(repc[:, :64], cw0_ref, cb0_ref),
        fc(repc[:, 64:], cw1_ref, cb1_ref),
        fc(reps[:, :64], sw0_ref, sb0_ref),
        fc(reps[:, 64:], sw1_ref, sb1_ref),
    ], axis=1)
    hg = jnp.maximum(
        jnp.dot(ind, f1w_ref[...], preferred_element_type=jnp.float32)
        + f1b_ref[...], 0.0)
    out_ref[...] = (jnp.dot(hg, f2w_ref[...],
                            preferred_element_type=jnp.float32)
                    + f2b_ref[...])


def kernel(chr_x, chr_edge_index, chr_x_batch, slv_x, slv_edge_index,
           slv_x_batch, pseudo_batch,
           chr_W00, chr_W01, chr_W10, chr_W11,
           slv_W00, slv_W01, slv_W10, slv_W11,
           cfc_w0, cfc_b0, cfc_w1, cfc_b1,
           sfc_w0, sfc_b0, sfc_w1, sfc_b1,
           fc1_w, fc1_b, fc2_w, fc2_b):
    del pseudo_batch
    mp = _msg_pass()
    pooled = {}
    for pre, x, ei, (W00, W01, W10, W11) in (
            ("chr", chr_x, chr_edge_index,
             (chr_W00, chr_W01, chr_W10, chr_W11)),
            ("slv", slv_x, slv_edge_index,
             (slv_W00, slv_W01, slv_W10, slv_W11))):
        src = ei[0]
        dst = ei[1]
        za, zb = mp(x, src, dst)
        g = _mid(za, zb, W00, W10, W01, W11)
        pa, pb = mp(g, src, dst)
        pooled[pre] = (pa, pb)

    out = pl.pallas_call(
        _head_kernel,
        out_shape=jax.ShapeDtypeStruct((B, 1), jnp.float32),
    )(pooled["chr"][0], pooled["chr"][1],
      pooled["slv"][0], pooled["slv"][1],
      chr_x_batch.reshape(1, N), slv_x_batch.reshape(1, N),
      cfc_w0, cfc_b0.reshape(1, -1), cfc_w1, cfc_b1.reshape(1, -1),
      sfc_w0, sfc_b0.reshape(1, -1), sfc_w1, sfc_b1.reshape(1, -1),
      fc1_w, fc1_b.reshape(1, -1), fc2_w, fc2_b.reshape(1, 1))
    return out


# speedup vs baseline: 15.2158x; 1.1633x over previous
import functools

import jax
import jax.numpy as jnp
from jax import lax
from jax.experimental import pallas as pl
from jax.experimental.pallas import tpu as pltpu
from jax.experimental.pallas import tpu_sc as plsc

N = 10000
E = 320000
B = 64
W = 128

_NT = 16
_EPT = E // (2 * _NT)
_K = 80
_NCHUNK = _EPT // _K
_ROWS_MAIN = 624
_ROWS_TAIL = N - 16 * _ROWS_MAIN


def _sc_msg_pass_fn():
    mesh = plsc.VectorSubcoreMesh(core_axis_name="c", subcore_axis_name="s")

    RI = 6
    RD = 4
    MOD = 12

    @functools.partial(
        pl.kernel,
        out_type=(jax.ShapeDtypeStruct((N, W), jnp.float32),
                  jax.ShapeDtypeStruct((N, W), jnp.float32)),
        mesh=mesh,
        scratch_types=(
            [pltpu.VMEM((_K,), jnp.int32)] * (2 * RI)
            + [pltpu.VMEM((_K, W), jnp.float32)] * RD
            + [pltpu.VMEM((8, W), jnp.float32),
               pltpu.VMEM_SHARED((N, W), jnp.float32)]
            + [pltpu.SemaphoreType.DMA] * (2 * RI + 2 * RD)
        ),
    )
    def msg_pass(h, src, dst, o_a, o_b, *scr):
        sis = scr[0:RI]
        dis = scr[RI:2 * RI]
        rowss = scr[2 * RI:2 * RI + RD]
        zbuf = scr[2 * RI + RD]
        acc = scr[2 * RI + RD + 1]
        sems = scr[2 * RI + RD + 2:]
        sss = sems[0:RI]
        dss = sems[RI:2 * RI]
        gss = sems[2 * RI:2 * RI + RD]
        css = sems[2 * RI + RD:]

        c = lax.axis_index("c")
        s = lax.axis_index("s")
        row0 = s * _ROWS_MAIN

        zero16 = jnp.zeros((16,), jnp.float32)

        def zb(i, _):
            r = i // (W // 16)
            col = (i % (W // 16)) * 16
            zbuf[r, pl.ds(col, 16)] = zero16
            return 0
        lax.fori_loop(0, 8 * W // 16, zb, 0)

        def zero_blk(i, _):
            pltpu.sync_copy(zbuf, acc.at[pl.ds(row0 + 8 * i, 8)])
            return 0
        lax.fori_loop(0, _ROWS_MAIN // 8, zero_blk, 0)

        @pl.when(s == _NT - 1)
        def _():
            def zero_tail(i, _):
                pltpu.sync_copy(zbuf, acc.at[pl.ds(16 * _ROWS_MAIN + 8 * i, 8)])
                return 0
            lax.fori_loop(0, _ROWS_TAIL // 8, zero_tail, 0)

        plsc.subcore_barrier()

        base0 = (c * _NT + s) * _EPT

        def load_idx(j, ri):
            off = base0 + j * _K
            pltpu.async_copy(src.at[pl.ds(off, _K)], sis[ri], sss[ri])
            pltpu.async_copy(dst.at[pl.ds(off, _K)], dis[ri], dss[ri])

        def start_gather(j, ri, rd, drain_prev):
            off = base0 + j * _K
            pltpu.make_async_copy(src.at[pl.ds(off, _K)], sis[ri],
                                  sss[ri]).wait()
            pltpu.make_async_copy(dst.at[pl.ds(off, _K)], dis[ri],
                                  dss[ri]).wait()

            @pl.when(drain_prev)
            def _():
                pltpu.make_async_copy(rowss[rd], acc.at[dis[ri]],
                                      css[rd]).wait()
            pltpu.async_copy(h.at[sis[ri]], rowss[rd], gss[rd])

        def finish(j, ri, rd):
            pltpu.make_async_copy(h.at[sis[ri]], rowss[rd], gss[rd]).wait()
            pltpu.async_copy(rowss[rd], acc.at[dis[ri]], css[rd], add=True)

        def step(j, r):
            @pl.when(j + 3 < _NCHUNK)
            def _():
                load_idx(j + 3, (r + 3) % RI)

            @pl.when(j + 2 < _NCHUNK)
            def _():
                start_gather(j + 2, (r + 2) % RI, (r + 2) % RD, j - 2 >= 0)
            finish(j, r % RI, r % RD)

        for jj in range(3):
            load_idx(jj, jj)
        start_gather(0, 0, 0, False)
        start_gather(1, 1, 1, False)

        def chunk(j, _):
            for r in range(MOD):
                @pl.when(j % MOD == r)
                def _(r=r):
                    step(j, r)
            return 0
        lax.fori_loop(0, _NCHUNK, chunk, 0)

        for jj in range(_NCHUNK - RD, _NCHUNK):
            pltpu.make_async_copy(rowss[jj % RD], acc.at[dis[jj % RI]],
                                  css[jj % RD]).wait()

        plsc.subcore_barrier()

        @pl.when(c == 0)
        def _():
            pltpu.sync_copy(acc.at[pl.ds(row0, _ROWS_MAIN)],
                            o_a.at[pl.ds(row0, _ROWS_MAIN)])

            @pl.when(s == _NT - 1)
            def _():
                pltpu.sync_copy(acc.at[pl.ds(16 * _ROWS_MAIN, _ROWS_TAIL)],
                                o_a.at[pl.ds(16 * _ROWS_MAIN, _ROWS_TAIL)])

        @pl.when(c == 1)
        def _():
            pltpu.sync_copy(acc.at[pl.ds(row0, _ROWS_MAIN)],
                            o_b.at[pl.ds(row0, _ROWS_MAIN)])

            @pl.when(s == _NT - 1)
            def _():
                pltpu.sync_copy(acc.at[pl.ds(16 * _ROWS_MAIN, _ROWS_TAIL)],
                                o_b.at[pl.ds(16 * _ROWS_MAIN, _ROWS_TAIL)])

    return msg_pass


_SC_CACHE = {}


def _msg_pass():
    if "mp" not in _SC_CACHE:
        _SC_CACHE["mp"] = _sc_msg_pass_fn()
    return _SC_CACHE["mp"]


def _mid_kernel(za_ref, zb_ref, w00_ref, w10_ref, w01_ref, w11_ref, g_ref):
    z = za_ref[...] + zb_ref[...]
    c0 = jnp.maximum(jnp.dot(z, w00_ref[...],
                             preferred_element_type=jnp.float32), 0.0)
    c1 = jnp.maximum(jnp.dot(z, w10_ref[...],
                             preferred_element_type=jnp.float32), 0.0)
    g_ref[...] = jnp.concatenate(
        [jnp.dot(c0, w01_ref[...], preferred_element_type=jnp.float32),
         jnp.dot(c1, w11_ref[...], preferred_element_type=jnp.float32)],
        axis=1)


def _mid(za, zb, w00, w10, w01, w11):
    return pl.pallas_call(
        _mid_kernel,
        out_shape=jax.ShapeDtypeStruct((N, W), jnp.float32),
    )(za, zb, w00, w10, w01, w11)


def _head_kernel(pca_ref, pcb_ref, psa_ref, psb_ref, cb_ref, sb_ref,
                 cw0_ref, cb0_ref, cw1_ref, cb1_ref,
                 sw0_ref, sb0_ref, sw1_ref, sb1_ref,
                 f1w_ref, f1b_ref, f2w_ref, f2b_ref, out_ref):
    iota_b = lax.broadcasted_iota(jnp.int32, (B, N), 0)
    pc = (cb_ref[...] == iota_b).astype(jnp.float32)
    ps = (sb_ref[...] == iota_b).astype(jnp.float32)

    m2c = jnp.maximum(pca_ref[...] + pcb_ref[...], 0.0)
    m2s = jnp.maximum(psa_ref[...] + psb_ref[...], 0.0)

    repc = jnp.dot(pc, m2c, preferred_element_type=jnp.float32)
    reps = jnp.dot(ps, m2s, preferred_element_type=jnp.float32)

    def fc(r, w_ref, b_ref):
        return jnp.maximum(
            jnp.dot(r, w_ref[...], preferred_element_type=jnp.float32)
            + b_ref[...], 0.0)

    ind = jnp.concatenate([
        fc(repc[:, :64], cw0_ref, cb0_ref),
        fc(repc[:, 64:], cw1_ref, cb1_ref),
        fc(reps[:, :64], sw0_ref, sb0_ref),
        fc(reps[:, 64:], sw1_ref, sb1_ref),
    ], axis=1)
    hg = jnp.maximum(
        jnp.dot(ind, f1w_ref[...], preferred_element_type=jnp.float32)
        + f1b_ref[...], 0.0)
    out_ref[...] = (jnp.dot(hg, f2w_ref[...],
                            preferred_element_type=jnp.float32)
                    + f2b_ref[...])


def kernel(chr_x, chr_edge_index, chr_x_batch, slv_x, slv_edge_index,
           slv_x_batch, pseudo_batch,
           chr_W00, chr_W01, chr_W10, chr_W11,
           slv_W00, slv_W01, slv_W10, slv_W11,
           cfc_w0, cfc_b0, cfc_w1, cfc_b1,
           sfc_w0, sfc_b0, sfc_w1, sfc_b1,
           fc1_w, fc1_b, fc2_w, fc2_b):
    del pseudo_batch
    mp = _msg_pass()
    pooled = {}
    for pre, x, ei, (W00, W01, W10, W11) in (
            ("chr", chr_x, chr_edge_index,
             (chr_W00, chr_W01, chr_W10, chr_W11)),
            ("slv", slv_x, slv_edge_index,
             (slv_W00, slv_W01, slv_W10, slv_W11))):
        src = ei[0]
        dst = ei[1]
        za, zb = mp(x, src, dst)
        g = _mid(za, zb, W00, W10, W01, W11)
        pa, pb = mp(g, src, dst)
        pooled[pre] = (pa, pb)

    out = pl.pallas_call(
        _head_kernel,
        out_shape=jax.ShapeDtypeStruct((B, 1), jnp.float32),
    )(pooled["chr"][0], pooled["chr"][1],
      pooled["slv"][0], pooled["slv"][1],
      chr_x_batch.reshape(1, N), slv_x_batch.reshape(1, N),
      cfc_w0, cfc_b0.reshape(1, -1), cfc_w1, cfc_b1.reshape(1, -1),
      sfc_w0, sfc_b0.reshape(1, -1), sfc_w1, sfc_b1.reshape(1, -1),
      fc1_w, fc1_b.reshape(1, -1), fc2_w, fc2_b.reshape(1, 1))
    return out


# speedup vs baseline: 16.7904x; 1.1035x over previous
import functools

import jax
import jax.numpy as jnp
from jax import lax
from jax.experimental import pallas as pl
from jax.experimental.pallas import tpu as pltpu
from jax.experimental.pallas import tpu_sc as plsc

N = 10000
E = 320000
B = 64
W = 128

_NT = 16
_EPT = E // _NT
_K = 80
_NCHUNK = _EPT // _K
_ROWS_MAIN = 624
_ROWS_TAIL = N - 16 * _ROWS_MAIN
_ZROWS = 48


def _sc_msg_pass_fn():
    mesh = plsc.VectorSubcoreMesh(core_axis_name="c", subcore_axis_name="s")

    RI = 6
    RD = 4
    MOD = 12

    @functools.partial(
        pl.kernel,
        out_type=(jax.ShapeDtypeStruct((N, W), jnp.float32),
                  jax.ShapeDtypeStruct((N, W), jnp.float32)),
        mesh=mesh,
        scratch_types=(
            [pltpu.VMEM((_K,), jnp.int32)] * (2 * RI)
            + [pltpu.VMEM((_K, W), jnp.float32)] * RD
            + [pltpu.VMEM((_ZROWS, W), jnp.float32),
               pltpu.VMEM_SHARED((N, W), jnp.float32)]
            + [pltpu.SemaphoreType.DMA] * (2 * RI + 2 * RD)
        ),
    )
    def msg_pass(h_c, h_s, src_c, dst_c, src_s, dst_s, o_c, o_s, *scr):
        sis = scr[0:RI]
        dis = scr[RI:2 * RI]
        rowss = scr[2 * RI:2 * RI + RD]
        zbuf = scr[2 * RI + RD]
        acc = scr[2 * RI + RD + 1]
        sems = scr[2 * RI + RD + 2:]
        sss = sems[0:RI]
        dss = sems[RI:2 * RI]
        gss = sems[2 * RI:2 * RI + RD]
        css = sems[2 * RI + RD:]

        c = lax.axis_index("c")
        s = lax.axis_index("s")
        row0 = s * _ROWS_MAIN
        base0 = s * _EPT

        def load_idx(src, dst, j, ri):
            off = base0 + j * _K
            pltpu.async_copy(src.at[pl.ds(off, _K)], sis[ri], sss[ri])
            pltpu.async_copy(dst.at[pl.ds(off, _K)], dis[ri], dss[ri])

        def start_gather(h, src, dst, j, ri, rd, drain_prev):
            off = base0 + j * _K
            pltpu.make_async_copy(src.at[pl.ds(off, _K)], sis[ri],
                                  sss[ri]).wait()
            pltpu.make_async_copy(dst.at[pl.ds(off, _K)], dis[ri],
                                  dss[ri]).wait()

            @pl.when(drain_prev)
            def _():
                pltpu.make_async_copy(rowss[rd], acc.at[dis[ri]],
                                      css[rd]).wait()
            pltpu.async_copy(h.at[sis[ri]], rowss[rd], gss[rd])

        def finish(h, ri, rd):
            pltpu.make_async_copy(h.at[sis[ri]], rowss[rd],
                                  gss[rd]).wait()
            pltpu.async_copy(rowss[rd], acc.at[dis[ri]], css[rd], add=True)

        def prologue(h, src, dst):
            for jj in range(3):
                load_idx(src, dst, jj, jj)
            start_gather(h, src, dst, 0, 0, 0, False)
            start_gather(h, src, dst, 1, 1, 1, False)

        def edge_loop(h, src, dst):
            def step(j, r):
                @pl.when(j + 3 < _NCHUNK)
                def _():
                    load_idx(src, dst, j + 3, (r + 3) % RI)

                @pl.when(j + 2 < _NCHUNK)
                def _():
                    start_gather(h, src, dst, j + 2, (r + 2) % RI,
                                 (r + 2) % RD, j - 2 >= 0)
                finish(h, r % RI, r % RD)

            def chunk(j, _):
                for r in range(MOD):
                    @pl.when(j % MOD == r)
                    def _(r=r):
                        step(j, r)
                return 0
            lax.fori_loop(0, _NCHUNK, chunk, 0)

            for jj in range(_NCHUNK - RD, _NCHUNK):
                pltpu.make_async_copy(rowss[jj % RD], acc.at[dis[jj % RI]],
                                      css[jj % RD]).wait()

        @pl.when(c == 0)
        def _():
            prologue(h_c, src_c, dst_c)

        @pl.when(c == 1)
        def _():
            prologue(h_s, src_s, dst_s)

        zero16 = jnp.zeros((16,), jnp.float32)

        def zb(i, _):
            r = i // (W // 16)
            col = (i % (W // 16)) * 16
            zbuf[r, pl.ds(col, 16)] = zero16
            return 0
        lax.fori_loop(0, _ZROWS * W // 16, zb, 0)

        def zero_blk(i, _):
            pltpu.sync_copy(zbuf, acc.at[pl.ds(row0 + _ZROWS * i, _ZROWS)])
            return 0
        lax.fori_loop(0, _ROWS_MAIN // _ZROWS, zero_blk, 0)

        @pl.when(s == _NT - 1)
        def _():
            pltpu.sync_copy(zbuf.at[pl.ds(0, _ROWS_TAIL)],
                            acc.at[pl.ds(16 * _ROWS_MAIN, _ROWS_TAIL)])

        plsc.subcore_barrier()

        @pl.when(c == 0)
        def _():
            edge_loop(h_c, src_c, dst_c)

        @pl.when(c == 1)
        def _():
            edge_loop(h_s, src_s, dst_s)

        plsc.subcore_barrier()

        def copy_out(o):
            pltpu.sync_copy(acc.at[pl.ds(row0, _ROWS_MAIN)],
                            o.at[pl.ds(row0, _ROWS_MAIN)])

            @pl.when(s == _NT - 1)
            def _():
                pltpu.sync_copy(acc.at[pl.ds(16 * _ROWS_MAIN, _ROWS_TAIL)],
                                o.at[pl.ds(16 * _ROWS_MAIN, _ROWS_TAIL)])

        @pl.when(c == 0)
        def _():
            copy_out(o_c)

        @pl.when(c == 1)
        def _():
            copy_out(o_s)

    return msg_pass


_SC_CACHE = {}


def _msg_pass():
    if "mp" not in _SC_CACHE:
        _SC_CACHE["mp"] = _sc_msg_pass_fn()
    return _SC_CACHE["mp"]


def _mid_kernel(zc_ref, zs_ref, cw00_ref, cw10_ref, cw01_ref, cw11_ref,
                sw00_ref, sw10_ref, sw01_ref, sw11_ref, gc_ref, gs_ref):
    def one(z_ref, w00_ref, w10_ref, w01_ref, w11_ref, g_ref):
        z = z_ref[...]
        c0 = jnp.maximum(jnp.dot(z, w00_ref[...],
                                 preferred_element_type=jnp.float32), 0.0)
        c1 = jnp.maximum(jnp.dot(z, w10_ref[...],
                                 preferred_element_type=jnp.float32), 0.0)
        g_ref[...] = jnp.concatenate(
            [jnp.dot(c0, w01_ref[...], preferred_element_type=jnp.float32),
             jnp.dot(c1, w11_ref[...], preferred_element_type=jnp.float32)],
            axis=1)

    one(zc_ref, cw00_ref, cw10_ref, cw01_ref, cw11_ref, gc_ref)
    one(zs_ref, sw00_ref, sw10_ref, sw01_ref, sw11_ref, gs_ref)


def _head_kernel(pc_ref, ps_ref, cb_ref, sb_ref,
                 cw0_ref, cb0_ref, cw1_ref, cb1_ref,
                 sw0_ref, sb0_ref, sw1_ref, sb1_ref,
                 f1w_ref, f1b_ref, f2w_ref, f2b_ref, out_ref):
    iota_b = lax.broadcasted_iota(jnp.int32, (B, N), 0)
    pc = (cb_ref[...] == iota_b).astype(jnp.float32)
    ps = (sb_ref[...] == iota_b).astype(jnp.float32)

    m2c = jnp.maximum(pc_ref[...], 0.0)
    m2s = jnp.maximum(ps_ref[...], 0.0)

    repc = jnp.dot(pc, m2c, preferred_element_type=jnp.float32)
    reps = jnp.dot(ps, m2s, preferred_element_type=jnp.float32)

    def fc(r, w_ref, b_ref):
        return jnp.maximum(
            jnp.dot(r, w_ref[...], preferred_element_type=jnp.float32)
            + b_ref[...], 0.0)

    ind = jnp.concatenate([
        fc(repc[:, :64], cw0_ref, cb0_ref),
        fc(repc[:, 64:], cw1_ref, cb1_ref),
        fc(reps[:, :64], sw0_ref, sb0_ref),
        fc(reps[:, 64:], sw1_ref, sb1_ref),
    ], axis=1)
    hg = jnp.maximum(
        jnp.dot(ind, f1w_ref[...], preferred_element_type=jnp.float32)
        + f1b_ref[...], 0.0)
    out_ref[...] = (jnp.dot(hg, f2w_ref[...],
                            preferred_element_type=jnp.float32)
                    + f2b_ref[...])


def kernel(chr_x, chr_edge_index, chr_x_batch, slv_x, slv_edge_index,
           slv_x_batch, pseudo_batch,
           chr_W00, chr_W01, chr_W10, chr_W11,
           slv_W00, slv_W01, slv_W10, slv_W11,
           cfc_w0, cfc_b0, cfc_w1, cfc_b1,
           sfc_w0, sfc_b0, sfc_w1, sfc_b1,
           fc1_w, fc1_b, fc2_w, fc2_b):
    del pseudo_batch
    mp = _msg_pass()
    csrc, cdst = chr_edge_index[0], chr_edge_index[1]
    ssrc, sdst = slv_edge_index[0], slv_edge_index[1]

    z_chr, z_slv = mp(chr_x, slv_x, csrc, cdst, ssrc, sdst)

    g_chr, g_slv = pl.pallas_call(
        _mid_kernel,
        out_shape=(jax.ShapeDtypeStruct((N, W), jnp.float32),
                   jax.ShapeDtypeStruct((N, W), jnp.float32)),
    )(z_chr, z_slv, chr_W00, chr_W10, chr_W01, chr_W11,
      slv_W00, slv_W10, slv_W01, slv_W11)

    p_chr, p_slv = mp(g_chr, g_slv, csrc, cdst, ssrc, sdst)

    out = pl.pallas_call(
        _head_kernel,
        out_shape=jax.ShapeDtypeStruct((B, 1), jnp.float32),
    )(p_chr, p_slv,
      chr_x_batch.reshape(1, N), slv_x_batch.reshape(1, N),
      cfc_w0, cfc_b0.reshape(1, -1), cfc_w1, cfc_b1.reshape(1, -1),
      sfc_w0, sfc_b0.reshape(1, -1), sfc_w1, sfc_b1.reshape(1, -1),
      fc1_w, fc1_b.reshape(1, -1), fc2_w, fc2_b.reshape(1, 1))
    return out
